# Initial kernel scaffold; baseline (speedup 1.0000x reference)
import functools

import jax
import jax.numpy as jnp
from jax import lax
from jax.experimental import pallas as pl
from jax.experimental.pallas import tpu as pltpu
from jax.experimental.pallas import tpu_sc as plsc

N_NODES = 10000
N_EDGES = 320000
D_IN = 128
H_REAL = 50
HP = 64
F_FEAT = 4
FHP = F_FEAT * HP
P_PAIRS = 1024
R_OUT = 3

NC = 2
NS = 16
NW = NC * NS

EPW = N_EDGES // NW
CHUNK = 80
NCHUNK = EPW // CHUNK
ROWS_PER_TILE = N_NODES // NS

PPW = P_PAIRS // NW


def _tc_stage_a(x_ref, w_ref, b_ref, ws_ref, bs_ref, y_ref, s_ref):
    xb = x_ref[...]
    y_ref[...] = jnp.dot(xb, w_ref[...], preferred_element_type=jnp.float32) + b_ref[...]
    s_ref[...] = jnp.dot(xb, ws_ref[...], preferred_element_type=jnp.float32) + bs_ref[...]


def _tc_stage_c(s0_ref, agg_ref, g_ref, b_ref, w1_ref, b1_ref, ws1_ref, bs1_ref,
                y_ref, s_ref):
    h = s0_ref[...] + agg_ref[0] + agg_ref[1]
    inv_h = 1.0 / H_REAL
    mu = jnp.sum(h, axis=-1, keepdims=True) * inv_h
    q = jnp.sum(h * h, axis=-1, keepdims=True) * inv_h
    var = q - mu * mu
    hn = (h - mu) * lax.rsqrt(var + 1e-5) * g_ref[...] + b_ref[...]
    y_ref[...] = jnp.dot(hn, w1_ref[...], preferred_element_type=jnp.float32) + b1_ref[...]
    s_ref[...] = jnp.dot(hn, ws1_ref[...], preferred_element_type=jnp.float32) + bs1_ref[...]


def _tc_stage_d(s1_ref, agg_ref, lw_ref, lb_ref, a1_ref, a2_ref, u1_ref, u2_ref):
    h2 = s1_ref[...] + agg_ref[0] + agg_ref[1]
    h3 = jnp.dot(h2, lw_ref[...], preferred_element_type=jnp.float32) + lb_ref[...]
    u1_ref[...] = jnp.dot(h3, a1_ref[...], preferred_element_type=jnp.float32)
    u2_ref[...] = jnp.dot(h3, a2_ref[...], preferred_element_type=jnp.float32)


def _tc_stage_f(g_ref, b_ref, w_ref, b2_ref, z_ref):
    t = g_ref[...] + b_ref[...]
    t = jnp.where(t >= 0, t, 0.01 * t)
    z_ref[...] = jnp.dot(t, w_ref[...], preferred_element_type=jnp.float32) + b2_ref[...]


def _sc_edge_body(y_hbm, src_hbm, dst_hbm, attr_hbm, zeros_hbm, out_hbm,
                  isrc, idst, attr_v, rows, msg, agg_sh, sem):
    c = lax.axis_index("c")
    s = lax.axis_index("s")
    wid = s * NC + c

    r0 = s * ROWS_PER_TILE
    pltpu.sync_copy(zeros_hbm.at[pl.ds(r0, ROWS_PER_TILE)],
                    agg_sh.at[pl.ds(r0, ROWS_PER_TILE)])
    plsc.subcore_barrier()

    base0 = wid * EPW

    def chunk_body(i, carry):
        base = base0 + i * CHUNK
        cp_s = pltpu.async_copy(src_hbm.at[pl.ds(base, CHUNK)], isrc, sem)
        cp_d = pltpu.async_copy(dst_hbm.at[pl.ds(base, CHUNK)], idst, sem)
        cp_a = pltpu.async_copy(attr_hbm.at[pl.ds(base * F_FEAT, CHUNK * F_FEAT)],
                                attr_v, sem)
        cp_s.wait()
        cp_d.wait()
        cp_a.wait()
        pltpu.async_copy(y_hbm.at[isrc], rows, sem).wait()

        def edge_body(e, carry2):
            ab = e * F_FEAT
            a0 = attr_v[ab]
            a1 = attr_v[ab + 1]
            a2 = attr_v[ab + 2]
            a3 = attr_v[ab + 3]
            for hc in range(HP // 16):
                off = hc * 16
                v = (a0 * rows[e, pl.ds(0 * HP + off, 16)]
                     + a1 * rows[e, pl.ds(1 * HP + off, 16)]
                     + a2 * rows[e, pl.ds(2 * HP + off, 16)]
                     + a3 * rows[e, pl.ds(3 * HP + off, 16)])
                msg[e, pl.ds(off, 16)] = v
            return carry2

        lax.fori_loop(0, CHUNK, edge_body, 0)
        pltpu.sync_copy(msg, agg_sh.at[idst], add=True)
        return carry

    lax.fori_loop(0, NCHUNK, chunk_body, 0)
    plsc.subcore_barrier()
    pltpu.sync_copy(agg_sh.at[pl.ds(r0, ROWS_PER_TILE)],
                    out_hbm.at[c, pl.ds(r0, ROWS_PER_TILE)])


def _sc_edge_pass(y, src, dst, attr_flat, zeros_nh):
    mesh = plsc.VectorSubcoreMesh(core_axis_name="c", subcore_axis_name="s")
    return pl.kernel(
        _sc_edge_body,
        out_type=jax.ShapeDtypeStruct((NC, N_NODES, HP), jnp.float32),
        mesh=mesh,
        scratch_types=[
            pltpu.VMEM((CHUNK,), jnp.int32),
            pltpu.VMEM((CHUNK,), jnp.int32),
            pltpu.VMEM((CHUNK * F_FEAT,), jnp.float32),
            pltpu.VMEM((CHUNK, FHP), jnp.float32),
            pltpu.VMEM((CHUNK, HP), jnp.float32),
            pltpu.VMEM_SHARED((N_NODES, HP), jnp.float32),
            pltpu.SemaphoreType.DMA,
        ],
    )(y, src, dst, attr_flat, zeros_nh)


def _sc_gather_body(u1_hbm, u2_hbm, e1_hbm, e2_hbm, out_hbm,
                    i1, i2, r1, r2, sem):
    c = lax.axis_index("c")
    s = lax.axis_index("s")
    wid = s * NC + c
    base = wid * PPW
    cp1 = pltpu.async_copy(e1_hbm.at[pl.ds(base, PPW)], i1, sem)
    cp2 = pltpu.async_copy(e2_hbm.at[pl.ds(base, PPW)], i2, sem)
    cp1.wait()
    cp2.wait()
    pltpu.async_copy(u1_hbm.at[i1], r1, sem).wait()
    pltpu.async_copy(u2_hbm.at[i2], r2, sem).wait()

    def row_body(e, carry):
        for hc in range(HP // 16):
            off = hc * 16
            r1[e, pl.ds(off, 16)] = r1[e, pl.ds(off, 16)] + r2[e, pl.ds(off, 16)]
        return carry

    lax.fori_loop(0, PPW, row_body, 0)
    pltpu.sync_copy(r1, out_hbm.at[pl.ds(base, PPW)])


def _sc_event_gather(u1, u2, e1, e2):
    mesh = plsc.VectorSubcoreMesh(core_axis_name="c", subcore_axis_name="s")
    return pl.kernel(
        _sc_gather_body,
        out_type=jax.ShapeDtypeStruct((P_PAIRS, HP), jnp.float32),
        mesh=mesh,
        scratch_types=[
            pltpu.VMEM((PPW,), jnp.int32),
            pltpu.VMEM((PPW,), jnp.int32),
            pltpu.VMEM((PPW, HP), jnp.float32),
            pltpu.VMEM((PPW, HP), jnp.float32),
            pltpu.SemaphoreType.DMA,
        ],
    )(u1, u2, e1, e2)


def _pad_to(a, shape):
    pads = [(0, t - s) for s, t in zip(a.shape, shape)]
    return jnp.pad(a, pads)


def kernel(x, edge_index, edge_attr, event1, event2, c0_W, c0_b, c0_Ws, c0_bs,
           c1_W, c1_b, c1_Ws, c1_bs, ln_g, ln_b, lin_W, lin_b, mp1_W, mp1_b,
           mp2_W, mp2_b):
    f32 = jnp.float32
    w0 = _pad_to(jnp.transpose(c0_W, (2, 0, 1)), (D_IN, F_FEAT, HP)).reshape(D_IN, FHP)
    b0 = _pad_to(c0_b, (F_FEAT, HP)).reshape(1, FHP)
    ws0 = _pad_to(c0_Ws.T, (D_IN, HP))
    bs0 = _pad_to(c0_bs, (HP,)).reshape(1, HP)
    w1 = _pad_to(jnp.transpose(c1_W, (2, 0, 1)), (HP, F_FEAT, HP)).reshape(HP, FHP)
    b1 = _pad_to(c1_b, (F_FEAT, HP)).reshape(1, FHP)
    ws1 = _pad_to(c1_Ws.T, (HP, HP))
    bs1 = _pad_to(c1_bs, (HP,)).reshape(1, HP)
    lng = _pad_to(ln_g, (HP,)).reshape(1, HP)
    lnb = _pad_to(ln_b, (HP,)).reshape(1, HP)
    lw = _pad_to(lin_W.T, (HP, HP))
    lb = _pad_to(lin_b, (HP,)).reshape(1, HP)
    a1 = _pad_to(mp1_W[:, :H_REAL].T, (HP, HP))
    a2 = _pad_to(mp1_W[:, H_REAL:].T, (HP, HP))
    mb1 = _pad_to(mp1_b, (HP,)).reshape(1, HP)
    w2 = _pad_to(mp2_W.T, (HP, 128))
    mb2 = _pad_to(mp2_b, (128,)).reshape(1, 128)

    src = edge_index[0]
    dst = edge_index[1]
    attr_flat = edge_attr.reshape(-1)
    zeros_nh = jnp.zeros((N_NODES, HP), dtype=f32)

    bn = 1000
    nb = N_NODES // bn

    y0, s0 = pl.pallas_call(
        _tc_stage_a,
        grid=(nb,),
        in_specs=[
            pl.BlockSpec((bn, D_IN), lambda i: (i, 0)),
            pl.BlockSpec((D_IN, FHP), lambda i: (0, 0)),
            pl.BlockSpec((1, FHP), lambda i: (0, 0)),
            pl.BlockSpec((D_IN, HP), lambda i: (0, 0)),
            pl.BlockSpec((1, HP), lambda i: (0, 0)),
        ],
        out_specs=[
            pl.BlockSpec((bn, FHP), lambda i: (i, 0)),
            pl.BlockSpec((bn, HP), lambda i: (i, 0)),
        ],
        out_shape=[
            jax.ShapeDtypeStruct((N_NODES, FHP), f32),
            jax.ShapeDtypeStruct((N_NODES, HP), f32),
        ],
    )(x, w0, b0, ws0, bs0)

    agg0 = _sc_edge_pass(y0, src, dst, attr_flat, zeros_nh)

    y1, s1 = pl.pallas_call(
        _tc_stage_c,
        grid=(nb,),
        in_specs=[
            pl.BlockSpec((bn, HP), lambda i: (i, 0)),
            pl.BlockSpec((NC, bn, HP), lambda i: (0, i, 0)),
            pl.BlockSpec((1, HP), lambda i: (0, 0)),
            pl.BlockSpec((1, HP), lambda i: (0, 0)),
            pl.BlockSpec((HP, FHP), lambda i: (0, 0)),
            pl.BlockSpec((1, FHP), lambda i: (0, 0)),
            pl.BlockSpec((HP, HP), lambda i: (0, 0)),
            pl.BlockSpec((1, HP), lambda i: (0, 0)),
        ],
        out_specs=[
            pl.BlockSpec((bn, FHP), lambda i: (i, 0)),
            pl.BlockSpec((bn, HP), lambda i: (i, 0)),
        ],
        out_shape=[
            jax.ShapeDtypeStruct((N_NODES, FHP), f32),
            jax.ShapeDtypeStruct((N_NODES, HP), f32),
        ],
    )(s0, agg0, lng, lnb, w1, b1, ws1, bs1)

    agg1 = _sc_edge_pass(y1, src, dst, attr_flat, zeros_nh)

    u1, u2 = pl.pallas_call(
        _tc_stage_d,
        grid=(nb,),
        in_specs=[
            pl.BlockSpec((bn, HP), lambda i: (i, 0)),
            pl.BlockSpec((NC, bn, HP), lambda i: (0, i, 0)),
            pl.BlockSpec((HP, HP), lambda i: (0, 0)),
            pl.BlockSpec((1, HP), lambda i: (0, 0)),
            pl.BlockSpec((HP, HP), lambda i: (0, 0)),
            pl.BlockSpec((HP, HP), lambda i: (0, 0)),
        ],
        out_specs=[
            pl.BlockSpec((bn, HP), lambda i: (i, 0)),
            pl.BlockSpec((bn, HP), lambda i: (i, 0)),
        ],
        out_shape=[
            jax.ShapeDtypeStruct((N_NODES, HP), f32),
            jax.ShapeDtypeStruct((N_NODES, HP), f32),
        ],
    )(s1, agg1, lw, lb, a1, a2)

    g = _sc_event_gather(u1, u2, event1, event2)

    z = pl.pallas_call(
        _tc_stage_f,
        grid=(1,),
        in_specs=[
            pl.BlockSpec((P_PAIRS, HP), lambda i: (0, 0)),
            pl.BlockSpec((1, HP), lambda i: (0, 0)),
            pl.BlockSpec((HP, 128), lambda i: (0, 0)),
            pl.BlockSpec((1, 128), lambda i: (0, 0)),
        ],
        out_specs=pl.BlockSpec((P_PAIRS, 128), lambda i: (0, 0)),
        out_shape=jax.ShapeDtypeStruct((P_PAIRS, 128), f32),
    )(g, mb1, w2, mb2)

    return z[:, :R_OUT]


# speedup vs baseline: 2.1663x; 2.1663x over previous
import functools

import jax
import jax.numpy as jnp
from jax import lax
from jax.experimental import pallas as pl
from jax.experimental.pallas import tpu as pltpu
from jax.experimental.pallas import tpu_sc as plsc

N_NODES = 10000
N_EDGES = 320000
D_IN = 128
H_REAL = 50
HP = 64
F_FEAT = 4
FHP = F_FEAT * HP
P_PAIRS = 1024
R_OUT = 3

NC = 2
NS = 16
NW = NC * NS

EPW = N_EDGES // NW
CHUNK = 80
NCHUNK = EPW // CHUNK
N_PAD = 10240
ROWS_PER_TILE = N_PAD // NS

PPW = P_PAIRS // NW


def _tc_stage_a(x_ref, w_ref, b_ref, ws_ref, bs_ref, y_ref, s_ref):
    xb = x_ref[...]
    y_ref[...] = jnp.dot(xb, w_ref[...], preferred_element_type=jnp.float32) + b_ref[...]
    s_ref[...] = jnp.dot(xb, ws_ref[...], preferred_element_type=jnp.float32) + bs_ref[...]


def _tc_stage_c(s0_ref, agg_ref, g_ref, b_ref, w1_ref, b1_ref, ws1_ref, bs1_ref,
                y_ref, s_ref):
    h = s0_ref[...] + agg_ref[0] + agg_ref[1]
    inv_h = 1.0 / H_REAL
    mu = jnp.sum(h, axis=-1, keepdims=True) * inv_h
    q = jnp.sum(h * h, axis=-1, keepdims=True) * inv_h
    var = q - mu * mu
    hn = (h - mu) * lax.rsqrt(var + 1e-5) * g_ref[...] + b_ref[...]
    y_ref[...] = jnp.dot(hn, w1_ref[...], preferred_element_type=jnp.float32) + b1_ref[...]
    s_ref[...] = jnp.dot(hn, ws1_ref[...], preferred_element_type=jnp.float32) + bs1_ref[...]


def _tc_stage_d(s1_ref, agg_ref, lw_ref, lb_ref, a12_ref, u_ref):
    h2 = s1_ref[...] + agg_ref[0] + agg_ref[1]
    h3 = jnp.dot(h2, lw_ref[...], preferred_element_type=jnp.float32) + lb_ref[...]
    u_ref[...] = jnp.dot(h3, a12_ref[...], preferred_element_type=jnp.float32)


def _tc_stage_f(g_ref, b_ref, w_ref, b2_ref, z_ref):
    t = g_ref[...] + b_ref[...]
    t = jnp.where(t >= 0, t, 0.01 * t)
    z_ref[...] = jnp.dot(t, w_ref[...], preferred_element_type=jnp.float32) + b2_ref[...]


def _sc_edge_body(y_hbm, src_hbm, dst_hbm, attr_hbm, zeros_hbm, out_hbm,
                  isrc, idst, attr_v, rows, msg, agg_sh, sem):
    c = lax.axis_index("c")
    s = lax.axis_index("s")
    wid = s * NC + c

    r0 = s * ROWS_PER_TILE
    pltpu.sync_copy(zeros_hbm.at[pl.ds(r0, ROWS_PER_TILE)],
                    agg_sh.at[pl.ds(r0, ROWS_PER_TILE)])
    plsc.subcore_barrier()

    base0 = wid * EPW

    def chunk_body(i, carry):
        base = base0 + i * CHUNK
        cp_s = pltpu.async_copy(src_hbm.at[pl.ds(base, CHUNK)], isrc, sem)
        cp_d = pltpu.async_copy(dst_hbm.at[pl.ds(base, CHUNK)], idst, sem)
        cp_a = pltpu.async_copy(attr_hbm.at[pl.ds(base * F_FEAT, CHUNK * F_FEAT)],
                                attr_v.at[pl.ds(0, CHUNK * F_FEAT)], sem)
        cp_s.wait()
        cp_d.wait()
        cp_a.wait()
        pltpu.async_copy(y_hbm.at[isrc], rows, sem).wait()

        def edge_body(e, carry2):
            av = attr_v[pl.ds(e * F_FEAT, 16)]
            a0 = av[0]
            a1 = av[1]
            a2 = av[2]
            a3 = av[3]
            for hc in range(HP // 16):
                off = hc * 16
                v = (a0 * rows[e, pl.ds(0 * HP + off, 16)]
                     + a1 * rows[e, pl.ds(1 * HP + off, 16)]
                     + a2 * rows[e, pl.ds(2 * HP + off, 16)]
                     + a3 * rows[e, pl.ds(3 * HP + off, 16)])
                msg[e, pl.ds(off, 16)] = v
            return carry2

        lax.fori_loop(0, CHUNK, edge_body, 0)
        pltpu.sync_copy(msg, agg_sh.at[idst], add=True)
        return carry

    lax.fori_loop(0, NCHUNK, chunk_body, 0)
    plsc.subcore_barrier()
    pltpu.sync_copy(agg_sh.at[pl.ds(r0, ROWS_PER_TILE)],
                    out_hbm.at[c, pl.ds(r0, ROWS_PER_TILE)])


def _sc_edge_pass(y, src, dst, attr_flat, zeros_nh):
    mesh = plsc.VectorSubcoreMesh(core_axis_name="c", subcore_axis_name="s")
    return pl.kernel(
        _sc_edge_body,
        out_type=jax.ShapeDtypeStruct((NC, N_PAD, HP), jnp.float32),
        mesh=mesh,
        scratch_types=[
            pltpu.VMEM((CHUNK,), jnp.int32),
            pltpu.VMEM((CHUNK,), jnp.int32),
            pltpu.VMEM((CHUNK * F_FEAT + 16,), jnp.float32),
            pltpu.VMEM((CHUNK, FHP), jnp.float32),
            pltpu.VMEM((CHUNK, HP), jnp.float32),
            pltpu.VMEM_SHARED((N_PAD, HP), jnp.float32),
            pltpu.SemaphoreType.DMA,
        ],
    )(y, src, dst, attr_flat, zeros_nh)


def _sc_gather_body(u_hbm, e1_hbm, e2_hbm, out_hbm,
                    i1, i2, r1, r2, g_v, sem):
    c = lax.axis_index("c")
    s = lax.axis_index("s")
    wid = s * NC + c
    base = wid * PPW
    cp1 = pltpu.async_copy(e1_hbm.at[pl.ds(base, PPW)], i1, sem)
    cp2 = pltpu.async_copy(e2_hbm.at[pl.ds(base, PPW)], i2, sem)
    cp1.wait()
    cp2.wait()
    pltpu.async_copy(u_hbm.at[i1], r1, sem).wait()
    pltpu.async_copy(u_hbm.at[i2], r2, sem).wait()

    def row_body(e, carry):
        for hc in range(HP // 16):
            off = hc * 16
            g_v[e, pl.ds(off, 16)] = (r1[e, pl.ds(off, 16)]
                                      + r2[e, pl.ds(HP + off, 16)])
        return carry

    lax.fori_loop(0, PPW, row_body, 0)
    pltpu.sync_copy(g_v, out_hbm.at[pl.ds(base, PPW)])


def _sc_event_gather(u, e1, e2):
    mesh = plsc.VectorSubcoreMesh(core_axis_name="c", subcore_axis_name="s")
    return pl.kernel(
        _sc_gather_body,
        out_type=jax.ShapeDtypeStruct((P_PAIRS, HP), jnp.float32),
        mesh=mesh,
        scratch_types=[
            pltpu.VMEM((PPW,), jnp.int32),
            pltpu.VMEM((PPW,), jnp.int32),
            pltpu.VMEM((PPW, 2 * HP), jnp.float32),
            pltpu.VMEM((PPW, 2 * HP), jnp.float32),
            pltpu.VMEM((PPW, HP), jnp.float32),
            pltpu.SemaphoreType.DMA,
        ],
    )(u, e1, e2)


def _pad_to(a, shape):
    pads = [(0, t - s) for s, t in zip(a.shape, shape)]
    return jnp.pad(a, pads)


def kernel(x, edge_index, edge_attr, event1, event2, c0_W, c0_b, c0_Ws, c0_bs,
           c1_W, c1_b, c1_Ws, c1_bs, ln_g, ln_b, lin_W, lin_b, mp1_W, mp1_b,
           mp2_W, mp2_b):
    f32 = jnp.float32
    w0 = _pad_to(jnp.transpose(c0_W, (2, 0, 1)), (D_IN, F_FEAT, HP)).reshape(D_IN, FHP)
    b0 = _pad_to(c0_b, (F_FEAT, HP)).reshape(1, FHP)
    ws0 = _pad_to(c0_Ws.T, (D_IN, HP))
    bs0 = _pad_to(c0_bs, (HP,)).reshape(1, HP)
    w1 = _pad_to(jnp.transpose(c1_W, (2, 0, 1)), (HP, F_FEAT, HP)).reshape(HP, FHP)
    b1 = _pad_to(c1_b, (F_FEAT, HP)).reshape(1, FHP)
    ws1 = _pad_to(c1_Ws.T, (HP, HP))
    bs1 = _pad_to(c1_bs, (HP,)).reshape(1, HP)
    lng = _pad_to(ln_g, (HP,)).reshape(1, HP)
    lnb = _pad_to(ln_b, (HP,)).reshape(1, HP)
    lw = _pad_to(lin_W.T, (HP, HP))
    lb = _pad_to(lin_b, (HP,)).reshape(1, HP)
    a12 = jnp.concatenate([_pad_to(mp1_W[:, :H_REAL].T, (HP, HP)),
                           _pad_to(mp1_W[:, H_REAL:].T, (HP, HP))], axis=1)
    mb1 = _pad_to(mp1_b, (HP,)).reshape(1, HP)
    w2 = _pad_to(mp2_W.T, (HP, 128))
    mb2 = _pad_to(mp2_b, (128,)).reshape(1, 128)

    src = edge_index[0]
    dst = edge_index[1]
    attr_flat = edge_attr.reshape(-1)
    zeros_nh = jnp.zeros((N_PAD, HP), dtype=f32)

    bn = 1000
    nb = N_NODES // bn

    y0, s0 = pl.pallas_call(
        _tc_stage_a,
        grid=(nb,),
        in_specs=[
            pl.BlockSpec((bn, D_IN), lambda i: (i, 0)),
            pl.BlockSpec((D_IN, FHP), lambda i: (0, 0)),
            pl.BlockSpec((1, FHP), lambda i: (0, 0)),
            pl.BlockSpec((D_IN, HP), lambda i: (0, 0)),
            pl.BlockSpec((1, HP), lambda i: (0, 0)),
        ],
        out_specs=[
            pl.BlockSpec((bn, FHP), lambda i: (i, 0)),
            pl.BlockSpec((bn, HP), lambda i: (i, 0)),
        ],
        out_shape=[
            jax.ShapeDtypeStruct((N_NODES, FHP), f32),
            jax.ShapeDtypeStruct((N_NODES, HP), f32),
        ],
    )(x, w0, b0, ws0, bs0)

    agg0 = _sc_edge_pass(y0, src, dst, attr_flat, zeros_nh)

    y1, s1 = pl.pallas_call(
        _tc_stage_c,
        grid=(nb,),
        in_specs=[
            pl.BlockSpec((bn, HP), lambda i: (i, 0)),
            pl.BlockSpec((NC, bn, HP), lambda i: (0, i, 0)),
            pl.BlockSpec((1, HP), lambda i: (0, 0)),
            pl.BlockSpec((1, HP), lambda i: (0, 0)),
            pl.BlockSpec((HP, FHP), lambda i: (0, 0)),
            pl.BlockSpec((1, FHP), lambda i: (0, 0)),
            pl.BlockSpec((HP, HP), lambda i: (0, 0)),
            pl.BlockSpec((1, HP), lambda i: (0, 0)),
        ],
        out_specs=[
            pl.BlockSpec((bn, FHP), lambda i: (i, 0)),
            pl.BlockSpec((bn, HP), lambda i: (i, 0)),
        ],
        out_shape=[
            jax.ShapeDtypeStruct((N_NODES, FHP), f32),
            jax.ShapeDtypeStruct((N_NODES, HP), f32),
        ],
    )(s0, agg0, lng, lnb, w1, b1, ws1, bs1)

    agg1 = _sc_edge_pass(y1, src, dst, attr_flat, zeros_nh)

    u = pl.pallas_call(
        _tc_stage_d,
        grid=(nb,),
        in_specs=[
            pl.BlockSpec((bn, HP), lambda i: (i, 0)),
            pl.BlockSpec((NC, bn, HP), lambda i: (0, i, 0)),
            pl.BlockSpec((HP, HP), lambda i: (0, 0)),
            pl.BlockSpec((1, HP), lambda i: (0, 0)),
            pl.BlockSpec((HP, 2 * HP), lambda i: (0, 0)),
        ],
        out_specs=pl.BlockSpec((bn, 2 * HP), lambda i: (i, 0)),
        out_shape=jax.ShapeDtypeStruct((N_NODES, 2 * HP), f32),
    )(s1, agg1, lw, lb, a12)

    g = _sc_event_gather(u, event1, event2)

    z = pl.pallas_call(
        _tc_stage_f,
        grid=(1,),
        in_specs=[
            pl.BlockSpec((P_PAIRS, HP), lambda i: (0, 0)),
            pl.BlockSpec((1, HP), lambda i: (0, 0)),
            pl.BlockSpec((HP, 128), lambda i: (0, 0)),
            pl.BlockSpec((1, 128), lambda i: (0, 0)),
        ],
        out_specs=pl.BlockSpec((P_PAIRS, 128), lambda i: (0, 0)),
        out_shape=jax.ShapeDtypeStruct((P_PAIRS, 128), f32),
    )(g, mb1, w2, mb2)

    return z[:, :R_OUT]
